# baseline (speedup 1.0000x reference)
import functools

import jax
import jax.numpy as jnp
from jax import lax
from jax.experimental import pallas as pl
from jax.experimental.pallas import tpu as pltpu
from jax.experimental.pallas import tpu_sc as plsc


def _make_mem_swap_body(blk, wcols, iblk):
    def body(t_ref, mem_ref, win_ref, idx_ref, ind_ref, omem_ref, oidx_ref):
        i = pl.program_id(0)
        omem_ref[...] = mem_ref[...]
        oidx_ref[...] = idx_ref[...]
        off = t_ref[0] * wcols - i * blk

        @pl.when((off >= 0) & (off + wcols <= blk))
        def _():
            omem_ref[:, pl.ds(off, wcols)] = win_ref[...]

        ioff = t_ref[0] * wcols - i * iblk

        @pl.when((ioff >= 0) & (ioff + wcols <= iblk))
        def _():
            oidx_ref[pl.ds(ioff, wcols)] = ind_ref[...]
    return body


def _mem_update_t(mem_t, win_t, idx, ind, t, blk):
    d, wcols = win_t.shape
    n = mem_t.shape[1] // blk
    iblk = idx.shape[0] // n
    return pl.pallas_call(
        _make_mem_swap_body(blk, wcols, iblk),
        grid=(n,),
        in_specs=[
            pl.BlockSpec(memory_space=pltpu.SMEM),
            pl.BlockSpec((d, blk), lambda i: (0, i)),
            pl.BlockSpec((d, wcols), lambda i: (0, 0)),
            pl.BlockSpec((iblk,), lambda i: (i,)),
            pl.BlockSpec((wcols,), lambda i: (0,)),
        ],
        out_specs=[
            pl.BlockSpec((d, blk), lambda i: (0, i)),
            pl.BlockSpec((iblk,), lambda i: (i,)),
        ],
        out_shape=[
            jax.ShapeDtypeStruct(mem_t.shape, mem_t.dtype),
            jax.ShapeDtypeStruct(idx.shape, idx.dtype),
        ],
        compiler_params=pltpu.CompilerParams(dimension_semantics=("parallel",)),
    )(t, mem_t, win_t, idx, ind)


def _make_pos_swap_body(blk, wcols):
    def body(t_ref, mem_ref, win_ref, idx_ref, ind_ref,
             omem_ref, oidx_ref, otab_ref):
        i = pl.program_id(0)
        omem_ref[...] = mem_ref[...]
        otab_ref[...] = mem_ref[...].T
        oidx_ref[...] = idx_ref[...]
        off = t_ref[0] * wcols - i * blk

        @pl.when((off >= 0) & (off + wcols <= blk))
        def _():
            omem_ref[:, pl.ds(off, wcols)] = win_ref[...]
            otab_ref[pl.ds(off, wcols), :] = win_ref[...].T
            oidx_ref[pl.ds(off, wcols)] = ind_ref[...]
    return body


def _pos_update_t(mem_t, win_t, idx, ind, t, blk):
    d, wcols = win_t.shape
    p = mem_t.shape[1]
    n = p // blk
    return pl.pallas_call(
        _make_pos_swap_body(blk, wcols),
        grid=(n,),
        in_specs=[
            pl.BlockSpec(memory_space=pltpu.SMEM),
            pl.BlockSpec((d, blk), lambda i: (0, i)),
            pl.BlockSpec((d, wcols), lambda i: (0, 0)),
            pl.BlockSpec((blk,), lambda i: (i,)),
            pl.BlockSpec((wcols,), lambda i: (0,)),
        ],
        out_specs=[
            pl.BlockSpec((d, blk), lambda i: (0, i)),
            pl.BlockSpec((blk,), lambda i: (i,)),
            pl.BlockSpec((blk, d), lambda i: (i, 0)),
        ],
        out_shape=[
            jax.ShapeDtypeStruct(mem_t.shape, mem_t.dtype),
            jax.ShapeDtypeStruct(idx.shape, idx.dtype),
            jax.ShapeDtypeStruct((p, d), mem_t.dtype),
        ],
        compiler_params=pltpu.CompilerParams(dimension_semantics=("parallel",)),
    )(t, mem_t, win_t, idx, ind)


def _sc_gather(table, idx):
    info = plsc.get_sparse_core_info()
    nc = info.num_cores
    b = idx.shape[0]
    d = table.shape[1]
    bpw = b // nc
    mesh = plsc.ScalarSubcoreMesh(axis_name="core", num_cores=nc)

    @functools.partial(
        pl.kernel,
        out_type=jax.ShapeDtypeStruct((b, d), table.dtype),
        mesh=mesh,
        scratch_types=[
            pltpu.SMEM((bpw,), jnp.int32),
            pltpu.SemaphoreType.DMA,
            pltpu.SemaphoreType.DMA,
        ],
    )
    def k(table_hbm, idx_hbm, out_hbm, idx_s, sem_i, sem):
        cid = lax.axis_index("core")
        base = cid * bpw
        pltpu.async_copy(idx_hbm.at[pl.ds(base, bpw)], idx_s, sem_i).wait()

        @pl.loop(0, bpw)
        def _(r):
            j = idx_s[r]
            pltpu.make_async_copy(
                table_hbm.at[pl.ds(j, 1), :],
                out_hbm.at[pl.ds(base + r, 1), :],
                sem,
            ).start()

        pltpu.make_async_copy(
            table_hbm.at[pl.ds(0, bpw), :],
            out_hbm.at[pl.ds(base, bpw), :],
            sem,
        ).wait()

    return k(table, idx)


def kernel(mem_ref, mem_pos, Y_ref, Z, indices, idx_ref_buf, idx_pos_buf,
           pos_sampled_idx, step_rel):
    B, d = Y_ref.shape
    R = mem_ref.shape[0]
    P = mem_pos.shape[0]

    step = jnp.asarray(step_rel, jnp.int32)
    t_ref = jnp.reshape(((step * B) % R) // B, (1,))
    t_pos = jnp.reshape(((step * B) % P) // B, (1,))

    mem_pos_new_t, idx_pos_new, table_rm = _pos_update_t(
        mem_pos.T, Z.T, idx_pos_buf, indices, t_pos, blk=2 * B)

    mem_ref_new_t, idx_ref_new = _mem_update_t(
        mem_ref.T, Y_ref.T, idx_ref_buf, indices, t_ref, blk=4 * B)

    Z_pseudo = _sc_gather(table_rm, pos_sampled_idx)

    return (mem_ref_new_t.T, idx_ref_new, mem_pos_new_t.T, idx_pos_new,
            Z_pseudo)


# speedup vs baseline: 1.0046x; 1.0046x over previous
import functools

import jax
import jax.numpy as jnp
from jax import lax
from jax.experimental import pallas as pl
from jax.experimental.pallas import tpu as pltpu
from jax.experimental.pallas import tpu_sc as plsc


def _make_mem_swap_body(blk, wcols, iblk):
    def body(t_ref, mem_ref, win_ref, idx_ref, ind_ref, omem_ref, oidx_ref):
        i = pl.program_id(0)
        omem_ref[...] = mem_ref[...]
        oidx_ref[...] = idx_ref[...]
        off = t_ref[0] * wcols - i * blk

        @pl.when((off >= 0) & (off + wcols <= blk))
        def _():
            omem_ref[:, pl.ds(off, wcols)] = win_ref[...]

        ioff = t_ref[0] * wcols - i * iblk

        @pl.when((ioff >= 0) & (ioff + wcols <= iblk))
        def _():
            oidx_ref[pl.ds(ioff, wcols)] = ind_ref[...]
    return body


def _mem_update_t(mem_t, win_t, idx, ind, t, blk):
    d, wcols = win_t.shape
    n = mem_t.shape[1] // blk
    iblk = idx.shape[0] // n
    return pl.pallas_call(
        _make_mem_swap_body(blk, wcols, iblk),
        grid=(n,),
        in_specs=[
            pl.BlockSpec(memory_space=pltpu.SMEM),
            pl.BlockSpec((d, blk), lambda i: (0, i)),
            pl.BlockSpec((d, wcols), lambda i: (0, 0)),
            pl.BlockSpec((iblk,), lambda i: (i,)),
            pl.BlockSpec((wcols,), lambda i: (0,)),
        ],
        out_specs=[
            pl.BlockSpec((d, blk), lambda i: (0, i)),
            pl.BlockSpec((iblk,), lambda i: (i,)),
        ],
        out_shape=[
            jax.ShapeDtypeStruct(mem_t.shape, mem_t.dtype),
            jax.ShapeDtypeStruct(idx.shape, idx.dtype),
        ],
        compiler_params=pltpu.CompilerParams(dimension_semantics=("parallel",)),
    )(t, mem_t, win_t, idx, ind)


def _make_pos_swap_body(blk, wcols):
    def body(t_ref, mem_ref, win_ref, idx_ref, ind_ref,
             omem_ref, oidx_ref, otab_ref):
        i = pl.program_id(0)
        omem_ref[...] = mem_ref[...]
        otab_ref[...] = mem_ref[...].T
        oidx_ref[...] = idx_ref[...]
        off = t_ref[0] * wcols - i * blk

        @pl.when((off >= 0) & (off + wcols <= blk))
        def _():
            omem_ref[:, pl.ds(off, wcols)] = win_ref[...]
            otab_ref[pl.ds(off, wcols), :] = win_ref[...].T
            oidx_ref[pl.ds(off, wcols)] = ind_ref[...]
    return body


def _pos_update_t(mem_t, win_t, idx, ind, t, blk):
    d, wcols = win_t.shape
    p = mem_t.shape[1]
    n = p // blk
    return pl.pallas_call(
        _make_pos_swap_body(blk, wcols),
        grid=(n,),
        in_specs=[
            pl.BlockSpec(memory_space=pltpu.SMEM),
            pl.BlockSpec((d, blk), lambda i: (0, i)),
            pl.BlockSpec((d, wcols), lambda i: (0, 0)),
            pl.BlockSpec((blk,), lambda i: (i,)),
            pl.BlockSpec((wcols,), lambda i: (0,)),
        ],
        out_specs=[
            pl.BlockSpec((d, blk), lambda i: (0, i)),
            pl.BlockSpec((blk,), lambda i: (i,)),
            pl.BlockSpec((blk, d), lambda i: (i, 0)),
        ],
        out_shape=[
            jax.ShapeDtypeStruct(mem_t.shape, mem_t.dtype),
            jax.ShapeDtypeStruct(idx.shape, idx.dtype),
            jax.ShapeDtypeStruct((p, d), mem_t.dtype),
        ],
        compiler_params=pltpu.CompilerParams(dimension_semantics=("parallel",)),
    )(t, mem_t, win_t, idx, ind)


def _sc_gather(table, idx):
    info = plsc.get_sparse_core_info()
    nc = info.num_cores
    b = idx.shape[0]
    d = table.shape[1]
    bpw = b // nc
    mesh = plsc.ScalarSubcoreMesh(axis_name="core", num_cores=nc)

    @functools.partial(
        pl.kernel,
        out_type=jax.ShapeDtypeStruct((b, d), table.dtype),
        mesh=mesh,
        scratch_types=[
            pltpu.SMEM((bpw,), jnp.int32),
            pltpu.SemaphoreType.DMA,
            pltpu.SemaphoreType.DMA,
        ],
    )
    def k(table_hbm, idx_hbm, out_hbm, idx_s, sem_i, sem):
        cid = lax.axis_index("core")
        base = cid * bpw
        pltpu.async_copy(idx_hbm.at[pl.ds(base, bpw)], idx_s, sem_i).wait()

        @pl.loop(0, bpw)
        def _(r):
            j = idx_s[r]
            pltpu.make_async_copy(
                table_hbm.at[pl.ds(j, 1), :],
                out_hbm.at[pl.ds(base + r, 1), :],
                sem,
            ).start()

        pltpu.make_async_copy(
            table_hbm.at[pl.ds(0, bpw), :],
            out_hbm.at[pl.ds(base, bpw), :],
            sem,
        ).wait()

    return k(table, idx)


def kernel(mem_ref, mem_pos, Y_ref, Z, indices, idx_ref_buf, idx_pos_buf,
           pos_sampled_idx, step_rel):
    B, d = Y_ref.shape
    R = mem_ref.shape[0]
    P = mem_pos.shape[0]

    step = jnp.asarray(step_rel, jnp.int32)
    t_ref = jnp.reshape(((step * B) % R) // B, (1,))
    t_pos = jnp.reshape(((step * B) % P) // B, (1,))

    mem_pos_new_t, idx_pos_new, table_rm = _pos_update_t(
        mem_pos.T, Z.T, idx_pos_buf, indices, t_pos, blk=4 * B)

    mem_ref_new_t, idx_ref_new = _mem_update_t(
        mem_ref.T, Y_ref.T, idx_ref_buf, indices, t_ref, blk=4 * B)

    Z_pseudo = _sc_gather(table_rm, pos_sampled_idx)

    return (mem_ref_new_t.T, idx_ref_new, mem_pos_new_t.T, idx_pos_new,
            Z_pseudo)
